# baseline (speedup 1.0000x reference)
import functools
import jax
import jax.numpy as jnp
import numpy as np
from jax import lax
from jax.experimental import pallas as pl
from jax.experimental.pallas import tpu as pltpu

N = 10000
D = 128
E = 64000
BM = 400
NBLK = N // BM

REL_NAMES = ("d_t_dr", "d_m_dr", "ddi", "d_p", "ppi", "dr_t_d", "dr_m_d", "p_d")
SRC_T = (2, 2, 0, 2, 1, 0, 0, 1)
DST_T = (0, 0, 0, 1, 1, 2, 2, 2)
AREL = (0, 1, 3, 2, 5, 6, 4, 7)
ASRC = tuple(SRC_T[r] for r in AREL)


def _lut(i, table):
    out = jnp.int32(table[-1])
    for k in reversed(range(len(table) - 1)):
        out = jnp.where(i == k, jnp.int32(table[k]), out)
    return out


def _pool_body(h_ref, w_ref, b_ref, o_ref):
    t = jnp.dot(h_ref[0], w_ref[0], preferred_element_type=jnp.float32)
    o_ref[0] = jax.nn.relu(t + b_ref[0])


def _pool_stage(h_stack, wp, bp):
    return pl.pallas_call(
        _pool_body,
        grid=(NBLK, 8),
        in_specs=[
            pl.BlockSpec((1, BM, D), lambda m, g: (_lut(g, ASRC), m, 0)),
            pl.BlockSpec((1, D, D), lambda m, g: (_lut(g, AREL), 0, 0)),
            pl.BlockSpec((1, 1, D), lambda m, g: (_lut(g, AREL), 0, 0)),
        ],
        out_specs=pl.BlockSpec((1, BM, D), lambda m, g: (_lut(g, AREL), m, 0)),
        out_shape=jax.ShapeDtypeStruct((8, N, D), jnp.float32),
    )(h_stack, wp, bp)


def _combine_body(h_ref, n_ref, ws_ref, wn_ref, b_ref, o_ref):
    r = pl.program_id(1)

    @pl.when((r == 0) | (r == 3) | (r == 5))
    def _():
        o_ref[0] = jnp.zeros_like(o_ref[0])

    t = (jnp.dot(h_ref[0], ws_ref[0], preferred_element_type=jnp.float32)
         + jnp.dot(n_ref[0], wn_ref[0], preferred_element_type=jnp.float32)
         + b_ref[0])
    o_ref[0] += jax.nn.relu(t)


def _combine_stage(h_stack, neigh, ws, wn, b):
    return pl.pallas_call(
        _combine_body,
        grid=(NBLK, 8),
        in_specs=[
            pl.BlockSpec((1, BM, D), lambda m, r: (_lut(r, DST_T), m, 0)),
            pl.BlockSpec((1, BM, D), lambda m, r: (r, m, 0)),
            pl.BlockSpec((1, D, D), lambda m, r: (r, 0, 0)),
            pl.BlockSpec((1, D, D), lambda m, r: (r, 0, 0)),
            pl.BlockSpec((1, 1, D), lambda m, r: (r, 0, 0)),
        ],
        out_specs=pl.BlockSpec((1, BM, D), lambda m, r: (_lut(r, DST_T), m, 0)),
        out_shape=jax.ShapeDtypeStruct((3, N, D), jnp.float32),
    )(h_stack, neigh, ws, wn, b)


def _segment_max_sc(hp, edges):
    outs = []
    for r in range(8):
        src = edges[r][0]
        dst = edges[r][1]
        m = hp[r][src]
        seg = jax.ops.segment_max(m, dst, num_segments=N)
        deg = jax.ops.segment_sum(jnp.ones((E,), jnp.float32), dst, num_segments=N)
        outs.append(jnp.where((deg > 0)[:, None], seg, 0.0))
    return jnp.stack(outs)


def kernel(h_dr, h_p, h_d, params, edges_d_t_dr, edges_d_m_dr, edges_d_p,
           edges_dr_t_d, edges_dr_m_d, edges_p_d, edges_ddi, edges_ppi):
    edges_by_name = {
        "d_t_dr": edges_d_t_dr, "d_m_dr": edges_d_m_dr, "d_p": edges_d_p,
        "dr_t_d": edges_dr_t_d, "dr_m_d": edges_dr_m_d, "p_d": edges_p_d,
        "ddi": edges_ddi, "ppi": edges_ppi,
    }
    edges = [edges_by_name[n] for n in REL_NAMES]
    wp = jnp.stack([params[n]["Wp"] for n in REL_NAMES])
    bp = jnp.stack([params[n]["bp"] for n in REL_NAMES])[:, None, :]
    ws = jnp.stack([params[n]["Ws"] for n in REL_NAMES])
    wn = jnp.stack([params[n]["Wn"] for n in REL_NAMES])
    b = jnp.stack([params[n]["b"] for n in REL_NAMES])[:, None, :]

    h = jnp.stack([h_dr, h_p, h_d])
    outs = []
    for _layer in range(2):
        hp = _pool_stage(h, wp, bp)
        neigh = _segment_max_sc(hp, edges)
        h = _combine_stage(h, neigh, ws, wn, b)
        outs.append(h)
    h1, h2 = outs
    return (h1[0], h1[1], h2[0], h2[1])
